# baseline (speedup 1.0000x reference)
import functools

import jax
import jax.numpy as jnp
from jax import lax
from jax.experimental import pallas as pl
from jax.experimental.pallas import tpu as pltpu
from jax.experimental.pallas import tpu_sc as plsc

_BATCH = 16384
_D = 32
_L = 16
_NC = 2
_NS = 16
_NW = _NC * _NS
_BPW = _BATCH // _NW

_mesh = plsc.VectorSubcoreMesh(core_axis_name="c", subcore_axis_name="s")


@functools.partial(
    pl.kernel,
    mesh=_mesh,
    out_type=jax.ShapeDtypeStruct((_BATCH, _D), jnp.float32),
    scratch_types=[
        pltpu.VMEM((_BPW,), jnp.int32),
        pltpu.VMEM((_BPW, _D), jnp.float32),
        pltpu.VMEM((_BPW, _D), jnp.float32),
        pltpu.VMEM((_BPW, _D), jnp.float32),
        pltpu.SemaphoreType.DMA,
        pltpu.SemaphoreType.DMA,
        pltpu.SemaphoreType.DMA,
    ],
    compiler_params=pltpu.CompilerParams(use_tc_tiling_on_sc=False),
)
def _reparam_kernel(labels_hbm, mu_hbm, sigma_hbm, z_hbm, out_hbm,
                    idx_v, mu_v, sg_v, z_v, sem_mu, sem_sg, sem_z):
    wid = lax.axis_index("s") * _NC + lax.axis_index("c")
    base = wid * _BPW

    pltpu.sync_copy(labels_hbm.at[pl.ds(base, _BPW)], idx_v)
    cp_mu = pltpu.async_copy(mu_hbm.at[idx_v], mu_v, sem_mu)
    cp_sg = pltpu.async_copy(sigma_hbm.at[idx_v], sg_v, sem_sg)
    cp_z = pltpu.async_copy(z_hbm.at[pl.ds(base, _BPW)], z_v, sem_z)
    cp_sg.wait()
    cp_mu.wait()
    cp_z.wait()

    def body(i, carry):
        for h in range(_D // _L):
            sl = pl.ds(h * _L, _L)
            s = sg_v[i, sl]
            m = mu_v[i, sl]
            zz = z_v[i, sl]
            z_v[i, sl] = zz * jnp.exp(s * 0.5) + m
        return carry

    lax.fori_loop(0, _BPW, body, 0)

    pltpu.sync_copy(z_v, out_hbm.at[pl.ds(base, _BPW)])


def kernel(labels, mu_table, sigma_table, z):
    return _reparam_kernel(labels.astype(jnp.int32), mu_table, sigma_table, z)
